# baseline (speedup 1.0000x reference)
import functools

import jax
import jax.numpy as jnp
from jax import lax
from jax.experimental import pallas as pl
from jax.experimental.pallas import tpu as pltpu
from jax.experimental.pallas import tpu_sc as plsc

N = 8192
D = 1024
FF = 2048
ND = 4
NA = N // ND * 3
NNEWS = N // ND

_NC = 2
_NS = 16
_NW = _NC * _NS
_GCH = 48
_SCH = 32

_G_RPW = NA // _NW
_S_RPW = N // _NW
_NEWS_W = NNEWS // _S_RPW


def _sc_gather_body(x_hbm, idx_hbm, out_hbm, idx_v, rows_v, sg0, sg1, sw0, sw1):
    wid = lax.axis_index("s") * _NC + lax.axis_index("c")
    base = wid * _G_RPW
    pltpu.sync_copy(idx_hbm.at[wid], idx_v)
    sg = (sg0, sg1)
    sw = (sw0, sw1)
    nch = _G_RPW // _GCH
    g = [None] * nch
    w = [None] * nch
    g[0] = pltpu.async_copy(x_hbm.at[idx_v.at[0]], rows_v.at[0], sg[0])
    for c in range(nch):
        b = c & 1
        g[c].wait()
        if c + 1 < nch:
            if c >= 1:
                w[c - 1].wait()
            g[c + 1] = pltpu.async_copy(
                x_hbm.at[idx_v.at[c + 1]], rows_v.at[1 - b], sg[1 - b]
            )
        w[c] = pltpu.async_copy(
            rows_v.at[b], out_hbm.at[pl.ds(base + c * _GCH, _GCH)], sw[b]
        )
    w[nch - 2].wait()
    w[nch - 1].wait()


def _sc_gather(x, idx_a):
    return pl.kernel(
        _sc_gather_body,
        out_type=jax.ShapeDtypeStruct((NA, D), jnp.float32),
        mesh=plsc.VectorSubcoreMesh(core_axis_name="c", subcore_axis_name="s"),
        scratch_types=[
            pltpu.VMEM((_G_RPW // _GCH, _GCH), jnp.int32),
            pltpu.VMEM((2, _GCH, D), jnp.float32),
            pltpu.SemaphoreType.DMA,
            pltpu.SemaphoreType.DMA,
            pltpu.SemaphoreType.DMA,
            pltpu.SemaphoreType.DMA,
        ],
    )(x, idx_a.reshape(_NW, _G_RPW // _GCH, _GCH))


def _sc_scatter_body(x_hbm, y_hbm, idx_hbm, out_hbm, idx_v, rows_v, sr0, sr1, sw0, sw1):
    wid = lax.axis_index("s") * _NC + lax.axis_index("c")
    base = wid * _S_RPW
    pltpu.sync_copy(idx_hbm.at[wid], idx_v)
    sr = (sr0, sr1)
    sw = (sw0, sw1)
    nch = _S_RPW // _SCH

    def _pipe(read_chunk):
        r = [None] * nch
        w = [None] * nch
        r[0] = read_chunk(0, 0, sr[0])
        for c in range(nch):
            b = c & 1
            r[c].wait()
            if c + 1 < nch:
                if c >= 1:
                    w[c - 1].wait()
                r[c + 1] = read_chunk(c + 1, 1 - b, sr[1 - b])
            w[c] = pltpu.async_copy(rows_v.at[b], out_hbm.at[idx_v.at[c]], sw[b])
        w[nch - 2].wait()
        w[nch - 1].wait()

    @pl.when(wid < _NEWS_W)
    def _news():
        _pipe(lambda c, b, sem: pltpu.async_copy(x_hbm.at[idx_v.at[c]], rows_v.at[b], sem))

    @pl.when(wid >= _NEWS_W)
    def _adapter():
        _pipe(
            lambda c, b, sem: pltpu.async_copy(
                y_hbm.at[pl.ds(base - NNEWS + c * _SCH, _SCH)], rows_v.at[b], sem
            )
        )


def _sc_scatter(x, y, idx):
    return pl.kernel(
        _sc_scatter_body,
        out_type=jax.ShapeDtypeStruct((N, D), jnp.float32),
        mesh=plsc.VectorSubcoreMesh(core_axis_name="c", subcore_axis_name="s"),
        scratch_types=[
            pltpu.VMEM((_S_RPW // _SCH, _SCH), jnp.int32),
            pltpu.VMEM((2, _SCH, D), jnp.float32),
            pltpu.SemaphoreType.DMA,
            pltpu.SemaphoreType.DMA,
            pltpu.SemaphoreType.DMA,
            pltpu.SemaphoreType.DMA,
        ],
    )(x, y, idx.reshape(_NW, _S_RPW // _SCH, _SCH))


_TR = 1024
_NT = (N // ND) // _TR


def _tc_adapter_body(x_ref, w1_ref, b1_ref, w2_ref, b2_ref, g_ref, b_ref, o_ref):
    xi = x_ref[...]
    mu = jnp.mean(xi, axis=-1, keepdims=True)
    m2 = jnp.mean(jnp.square(xi), axis=-1, keepdims=True)
    sd = jnp.sqrt(jnp.maximum(m2 - jnp.square(mu), 0.0))
    scale = g_ref[0] / (sd + 1e-6)
    h = (scale * xi + (b_ref[0] - scale * mu)).astype(jnp.bfloat16)
    a = (
        jnp.maximum(
            jnp.dot(
                h,
                w1_ref[0].astype(jnp.bfloat16),
                preferred_element_type=jnp.float32,
            )
            + b1_ref[0],
            0.0,
        )
    ).astype(jnp.bfloat16)
    ff = jnp.dot(
        a,
        w2_ref[0].astype(jnp.bfloat16),
        preferred_element_type=jnp.float32,
    )
    o_ref[...] = (xi + ff) + b2_ref[0]


def _tc_adapter(xg, W1, b1, W2, b2, ln_g, ln_b):
    wmap = lambda d, r: (d, 0, 0)
    return pl.pallas_call(
        _tc_adapter_body,
        grid=(3, _NT),
        in_specs=[
            pl.BlockSpec((_TR, D), lambda d, r: (d * _NT + r, 0)),
            pl.BlockSpec((1, D, FF), wmap),
            pl.BlockSpec((1, 1, FF), wmap),
            pl.BlockSpec((1, FF, D), wmap),
            pl.BlockSpec((1, 1, D), wmap),
            pl.BlockSpec((1, 1, D), wmap),
            pl.BlockSpec((1, 1, D), wmap),
        ],
        out_specs=pl.BlockSpec((_TR, D), lambda d, r: (d * _NT + r, 0)),
        out_shape=jax.ShapeDtypeStruct((NA, D), jnp.float32),
        compiler_params=pltpu.CompilerParams(
            dimension_semantics=("arbitrary", "arbitrary"),
        ),
    )(
        xg,
        W1,
        b1.reshape(3, 1, FF),
        W2,
        b2.reshape(3, 1, D),
        ln_g.reshape(3, 1, D),
        ln_b.reshape(3, 1, D),
    )


def kernel(x, target_domain, W1, b1, W2, b2, ln_g, ln_b):
    idx = target_domain.reshape(N).astype(jnp.int32)
    xg = _sc_gather(x, idx[NNEWS:])
    y = _tc_adapter(xg, W1, b1, W2, b2, ln_g, ln_b)
    out = _sc_scatter(x, y, idx)
    return out


# speedup vs baseline: 1.0002x; 1.0002x over previous
import functools

import jax
import jax.numpy as jnp
from jax import lax
from jax.experimental import pallas as pl
from jax.experimental.pallas import tpu as pltpu
from jax.experimental.pallas import tpu_sc as plsc

N = 8192
D = 1024
FF = 2048
ND = 4
NA = N // ND * 3
NNEWS = N // ND

_NC = 2
_NS = 16
_NW = _NC * _NS
_GCH = 48
_SCH = 32

_G_RPW = NA // _NW
_S_RPW = N // _NW
_NEWS_W = NNEWS // _S_RPW


def _sc_gather_body(x_hbm, idx_hbm, out_hbm, idx_v, rows_v, sg0, sg1, sw0, sw1):
    wid = lax.axis_index("s") * _NC + lax.axis_index("c")
    base = wid * _G_RPW
    pltpu.sync_copy(idx_hbm.at[wid], idx_v)
    sg = (sg0, sg1)
    sw = (sw0, sw1)
    nch = _G_RPW // _GCH
    g = [None] * nch
    w = [None] * nch
    g[0] = pltpu.async_copy(x_hbm.at[idx_v.at[0]], rows_v.at[0], sg[0])
    for c in range(nch):
        b = c & 1
        g[c].wait()
        if c + 1 < nch:
            if c >= 1:
                w[c - 1].wait()
            g[c + 1] = pltpu.async_copy(
                x_hbm.at[idx_v.at[c + 1]], rows_v.at[1 - b], sg[1 - b]
            )
        w[c] = pltpu.async_copy(
            rows_v.at[b], out_hbm.at[pl.ds(base + c * _GCH, _GCH)], sw[b]
        )
    w[nch - 2].wait()
    w[nch - 1].wait()


def _sc_gather(x, idx_a):
    return pl.kernel(
        _sc_gather_body,
        out_type=jax.ShapeDtypeStruct((NA, D), jnp.float32),
        mesh=plsc.VectorSubcoreMesh(core_axis_name="c", subcore_axis_name="s"),
        scratch_types=[
            pltpu.VMEM((_G_RPW // _GCH, _GCH), jnp.int32),
            pltpu.VMEM((2, _GCH, D), jnp.float32),
            pltpu.SemaphoreType.DMA,
            pltpu.SemaphoreType.DMA,
            pltpu.SemaphoreType.DMA,
            pltpu.SemaphoreType.DMA,
        ],
    )(x, idx_a.reshape(_NW, _G_RPW // _GCH, _GCH))


def _sc_scatter_body(x_hbm, y_hbm, idx_hbm, out_hbm, idx_v, rows_v, sr0, sr1, sw0, sw1):
    wid = lax.axis_index("s") * _NC + lax.axis_index("c")
    base = wid * _S_RPW
    pltpu.sync_copy(idx_hbm.at[wid], idx_v)
    sr = (sr0, sr1)
    sw = (sw0, sw1)
    nch = _S_RPW // _SCH

    def _pipe(read_chunk):
        r = [None] * nch
        w = [None] * nch
        r[0] = read_chunk(0, 0, sr[0])
        for c in range(nch):
            b = c & 1
            r[c].wait()
            if c + 1 < nch:
                if c >= 1:
                    w[c - 1].wait()
                r[c + 1] = read_chunk(c + 1, 1 - b, sr[1 - b])
            w[c] = pltpu.async_copy(rows_v.at[b], out_hbm.at[idx_v.at[c]], sw[b])
        w[nch - 2].wait()
        w[nch - 1].wait()

    @pl.when(wid < _NEWS_W)
    def _news():
        _pipe(lambda c, b, sem: pltpu.async_copy(x_hbm.at[idx_v.at[c]], rows_v.at[b], sem))

    @pl.when(wid >= _NEWS_W)
    def _adapter():
        _pipe(
            lambda c, b, sem: pltpu.async_copy(
                y_hbm.at[pl.ds(base - NNEWS + c * _SCH, _SCH)], rows_v.at[b], sem
            )
        )


def _sc_scatter(x, y, idx):
    return pl.kernel(
        _sc_scatter_body,
        out_type=jax.ShapeDtypeStruct((N, D), jnp.float32),
        mesh=plsc.VectorSubcoreMesh(core_axis_name="c", subcore_axis_name="s"),
        scratch_types=[
            pltpu.VMEM((_S_RPW // _SCH, _SCH), jnp.int32),
            pltpu.VMEM((2, _SCH, D), jnp.float32),
            pltpu.SemaphoreType.DMA,
            pltpu.SemaphoreType.DMA,
            pltpu.SemaphoreType.DMA,
            pltpu.SemaphoreType.DMA,
        ],
    )(x, y, idx.reshape(_NW, _S_RPW // _SCH, _SCH))


_TR = 1024
_NT = (N // ND) // _TR


def _tc_adapter_body(x_ref, w1_ref, b1_ref, w2_ref, b2_ref, g_ref, b_ref, o_ref):
    w1 = w1_ref[0].astype(jnp.bfloat16)
    w2 = w2_ref[0].astype(jnp.bfloat16)

    def _half(sl):
        xi = x_ref[sl, :]
        mu = jnp.mean(xi, axis=-1, keepdims=True)
        m2 = jnp.mean(jnp.square(xi), axis=-1, keepdims=True)
        sd = jnp.sqrt(jnp.maximum(m2 - jnp.square(mu), 0.0))
        scale = g_ref[0] / (sd + 1e-6)
        h = (scale * xi + (b_ref[0] - scale * mu)).astype(jnp.bfloat16)
        a = (
            jnp.maximum(
                jnp.dot(h, w1, preferred_element_type=jnp.float32) + b1_ref[0],
                0.0,
            )
        ).astype(jnp.bfloat16)
        ff = jnp.dot(a, w2, preferred_element_type=jnp.float32)
        o_ref[sl, :] = (xi + ff) + b2_ref[0]

    _half(pl.ds(0, _TR // 2))
    _half(pl.ds(_TR // 2, _TR // 2))


def _tc_adapter(xg, W1, b1, W2, b2, ln_g, ln_b):
    wmap = lambda d, r: (d, 0, 0)
    return pl.pallas_call(
        _tc_adapter_body,
        grid=(3, _NT),
        in_specs=[
            pl.BlockSpec((_TR, D), lambda d, r: (d * _NT + r, 0)),
            pl.BlockSpec((1, D, FF), wmap),
            pl.BlockSpec((1, 1, FF), wmap),
            pl.BlockSpec((1, FF, D), wmap),
            pl.BlockSpec((1, 1, D), wmap),
            pl.BlockSpec((1, 1, D), wmap),
            pl.BlockSpec((1, 1, D), wmap),
        ],
        out_specs=pl.BlockSpec((_TR, D), lambda d, r: (d * _NT + r, 0)),
        out_shape=jax.ShapeDtypeStruct((NA, D), jnp.float32),
        compiler_params=pltpu.CompilerParams(
            dimension_semantics=("arbitrary", "arbitrary"),
        ),
    )(
        xg,
        W1,
        b1.reshape(3, 1, FF),
        W2,
        b2.reshape(3, 1, D),
        ln_g.reshape(3, 1, D),
        ln_b.reshape(3, 1, D),
    )


def kernel(x, target_domain, W1, b1, W2, b2, ln_g, ln_b):
    idx = target_domain.reshape(N).astype(jnp.int32)
    xg = _sc_gather(x, idx[NNEWS:])
    y = _tc_adapter(xg, W1, b1, W2, b2, ln_g, ln_b)
    out = _sc_scatter(x, y, idx)
    return out


# speedup vs baseline: 1.0290x; 1.0288x over previous
import functools

import jax
import jax.numpy as jnp
from jax import lax
from jax.experimental import pallas as pl
from jax.experimental.pallas import tpu as pltpu
from jax.experimental.pallas import tpu_sc as plsc

N = 8192
D = 1024
FF = 2048
ND = 4
NA = N // ND * 3
NNEWS = N // ND

_NC = 2
_NS = 16
_NW = _NC * _NS
_CHUNK = 64

_G_RPW = NA // _NW
_S_RPW = N // _NW
_NEWS_W = NNEWS // _S_RPW


def _sc_gather_body(x_hbm, idx_hbm, out_hbm, idx_v, rows_v, sem, *, rpw):
    wid = lax.axis_index("s") * _NC + lax.axis_index("c")
    base = wid * rpw
    for c in range(rpw // _CHUNK):
        off = base + c * _CHUNK
        pltpu.sync_copy(idx_hbm.at[pl.ds(off, _CHUNK)], idx_v)
        pltpu.async_copy(x_hbm.at[idx_v], rows_v, sem).wait()
        pltpu.sync_copy(rows_v, out_hbm.at[pl.ds(off, _CHUNK)])


def _sc_gather(x, idx_a, nrows):
    body = functools.partial(_sc_gather_body, rpw=nrows // _NW)
    return pl.kernel(
        body,
        out_type=jax.ShapeDtypeStruct((nrows, D), jnp.float32),
        mesh=plsc.VectorSubcoreMesh(core_axis_name="c", subcore_axis_name="s"),
        scratch_types=[
            pltpu.VMEM((_CHUNK,), jnp.int32),
            pltpu.VMEM((_CHUNK, D), jnp.float32),
            pltpu.SemaphoreType.DMA,
        ],
    )(x, idx_a)


def _sc_scatter_body(x_hbm, y1_hbm, y23_hbm, idx_hbm, out_hbm, idx_v, rows_v, sem):
    wid = lax.axis_index("s") * _NC + lax.axis_index("c")
    base = wid * _S_RPW

    def _move(read_chunk):
        for c in range(_S_RPW // _CHUNK):
            off = base + c * _CHUNK
            pltpu.sync_copy(idx_hbm.at[pl.ds(off, _CHUNK)], idx_v)
            read_chunk(off)
            pltpu.async_copy(rows_v, out_hbm.at[idx_v], sem).wait()

    @pl.when(wid < 8)
    def _news():
        _move(lambda off: pltpu.async_copy(x_hbm.at[idx_v], rows_v, sem).wait())

    @pl.when(jnp.logical_and(wid >= 8, wid < 16))
    def _dom1():
        _move(lambda off: pltpu.sync_copy(y1_hbm.at[pl.ds(off - 2048, _CHUNK)], rows_v))

    @pl.when(wid >= 16)
    def _dom23():
        _move(lambda off: pltpu.sync_copy(y23_hbm.at[pl.ds(off - 4096, _CHUNK)], rows_v))


def _sc_scatter(x, y1, y23, idx):
    return pl.kernel(
        _sc_scatter_body,
        out_type=jax.ShapeDtypeStruct((N, D), jnp.float32),
        mesh=plsc.VectorSubcoreMesh(core_axis_name="c", subcore_axis_name="s"),
        scratch_types=[
            pltpu.VMEM((_CHUNK,), jnp.int32),
            pltpu.VMEM((_CHUNK, D), jnp.float32),
            pltpu.SemaphoreType.DMA,
        ],
    )(x, y1, y23, idx)


_TR = 1024
_NT = (N // ND) // _TR


def _tc_adapter_body(x_ref, w1_ref, b1_ref, w2_ref, b2_ref, g_ref, b_ref, o_ref):
    xi = x_ref[...]
    mu = jnp.mean(xi, axis=-1, keepdims=True)
    m2 = jnp.mean(jnp.square(xi), axis=-1, keepdims=True)
    sd = jnp.sqrt(jnp.maximum(m2 - jnp.square(mu), 0.0))
    scale = g_ref[0] / (sd + 1e-6)
    h = (scale * xi + (b_ref[0] - scale * mu)).astype(jnp.bfloat16)
    a = (
        jnp.maximum(
            jnp.dot(
                h,
                w1_ref[0].astype(jnp.bfloat16),
                preferred_element_type=jnp.float32,
            )
            + b1_ref[0],
            0.0,
        )
    ).astype(jnp.bfloat16)
    ff = jnp.dot(
        a,
        w2_ref[0].astype(jnp.bfloat16),
        preferred_element_type=jnp.float32,
    )
    o_ref[...] = (xi + ff) + b2_ref[0]


def _tc_adapter(xg, W1, b1, W2, b2, ln_g, ln_b, dbase, ndom):
    wmap = lambda d, r: (d + dbase, 0, 0)
    return pl.pallas_call(
        _tc_adapter_body,
        grid=(ndom, _NT),
        in_specs=[
            pl.BlockSpec((_TR, D), lambda d, r: (d * _NT + r, 0)),
            pl.BlockSpec((1, D, FF), wmap),
            pl.BlockSpec((1, 1, FF), wmap),
            pl.BlockSpec((1, FF, D), wmap),
            pl.BlockSpec((1, 1, D), wmap),
            pl.BlockSpec((1, 1, D), wmap),
            pl.BlockSpec((1, 1, D), wmap),
        ],
        out_specs=pl.BlockSpec((_TR, D), lambda d, r: (d * _NT + r, 0)),
        out_shape=jax.ShapeDtypeStruct((ndom * 2048, D), jnp.float32),
        compiler_params=pltpu.CompilerParams(
            dimension_semantics=("arbitrary", "arbitrary"),
        ),
    )(
        xg,
        W1,
        b1.reshape(3, 1, FF),
        W2,
        b2.reshape(3, 1, D),
        ln_g.reshape(3, 1, D),
        ln_b.reshape(3, 1, D),
    )


def kernel(x, target_domain, W1, b1, W2, b2, ln_g, ln_b):
    idx = target_domain.reshape(N).astype(jnp.int32)
    xg1 = _sc_gather(x, idx[2048:4096], 2048)
    xg23 = _sc_gather(x, idx[4096:], 4096)
    y1 = _tc_adapter(xg1, W1, b1, W2, b2, ln_g, ln_b, 0, 1)
    y23 = _tc_adapter(xg23, W1, b1, W2, b2, ln_g, ln_b, 1, 2)
    out = _sc_scatter(x, y1, y23, idx)
    return out
